# Initial kernel scaffold; baseline (speedup 1.0000x reference)
import functools

import jax
import jax.numpy as jnp
from jax import lax
from jax.experimental import pallas as pl
from jax.experimental.pallas import tpu as pltpu
from jax.experimental.pallas import tpu_sc as plsc

NW = 32
NCORE = 2
NSUB = 16
CHUNK = 80


def _make_edge_pass(n_nodes, n_edges, with_deg):
  per_tile = n_edges // NW
  n_chunks = per_tile // CHUNK
  rows_per_tile = n_nodes // NSUB
  mesh = plsc.VectorSubcoreMesh(core_axis_name="c", subcore_axis_name="s")

  out_type = [jax.ShapeDtypeStruct((NCORE * n_nodes, 16), jnp.float32)]
  scratch = [
      pltpu.VMEM((per_tile,), jnp.int32),
      pltpu.VMEM((per_tile,), jnp.int32),
      pltpu.VMEM((per_tile,), jnp.float32),
      pltpu.VMEM((CHUNK,), jnp.int32),
      pltpu.VMEM((CHUNK, 32), jnp.float32),
      pltpu.VMEM((CHUNK, 16), jnp.float32),
      pltpu.VMEM((n_nodes // NSUB, 16), jnp.float32),
      pltpu.VMEM_SHARED((n_nodes, 16), jnp.float32),
      pltpu.SemaphoreType.DMA,
  ]
  if with_deg:
    out_type.append(jax.ShapeDtypeStruct((NCORE * n_nodes,), jnp.float32))
    scratch += [
        pltpu.VMEM((CHUNK,), jnp.float32),
        pltpu.VMEM((n_nodes,), jnp.float32),
        pltpu.VMEM_SHARED((n_nodes,), jnp.float32),
    ]

  def body(ab_hbm, src_hbm, dst_hbm, p_hbm, acc_out, *rest):
    if with_deg:
      (deg_out, src_t, dst_t, p_t, idxw, rows_v, msg_v, zb, acc_sh, sem,
       ones_v, zdeg, deg_sh) = rest
    else:
      (src_t, dst_t, p_t, idxw, rows_v, msg_v, zb, acc_sh, sem) = rest
      deg_out = ones_v = zdeg = deg_sh = None

    cid = lax.axis_index("c")
    sid = lax.axis_index("s")
    wid = cid * NSUB + sid
    ebase = wid * per_tile

    pltpu.sync_copy(src_hbm.at[pl.ds(ebase, per_tile)], src_t)
    pltpu.sync_copy(dst_hbm.at[pl.ds(ebase, per_tile)], dst_t)
    pltpu.sync_copy(p_hbm.at[pl.ds(ebase, per_tile)], p_t)

    @functools.partial(lax.fori_loop, 0, rows_per_tile, init_val=None)
    def _(i, _):
      zb[i, :] = jnp.zeros((16,), jnp.float32)

    pltpu.sync_copy(zb, acc_sh.at[pl.ds(sid * rows_per_tile, rows_per_tile)])
    if with_deg:
      for u in range(CHUNK // 16):
        ones_v[pl.ds(u * 16, 16)] = jnp.ones((16,), jnp.float32)

      @pl.when(sid == 0)
      def _():
        @functools.partial(lax.fori_loop, 0, n_nodes // 16, init_val=None)
        def _(i, _):
          zdeg[pl.ds(i * 16, 16)] = jnp.zeros((16,), jnp.float32)
        pltpu.sync_copy(zdeg, deg_sh)

    plsc.subcore_barrier()

    @functools.partial(lax.fori_loop, 0, n_chunks, init_val=None)
    def _(j, _):
      base = j * CHUNK
      for u in range(CHUNK // 16):
        idxw[pl.ds(u * 16, 16)] = dst_t[pl.ds(base + u * 16, 16)]
      pltpu.async_copy(
          ab_hbm.at[src_t.at[pl.ds(base, CHUNK)]], rows_v, sem).wait()

      @functools.partial(plsc.parallel_loop, 0, CHUNK, unroll=8)
      def _(i):
        p = p_t[base + i]
        p = jnp.minimum(jnp.maximum(p, 0.0), 1.0)
        a = rows_v[i, pl.ds(0, 16)]
        b = rows_v[i, pl.ds(16, 16)]
        msg_v[i, :] = a + p * (b - a)

      pltpu.sync_copy(msg_v, acc_sh.at[idxw], add=True)
      if with_deg:
        pltpu.sync_copy(ones_v, deg_sh.at[idxw], add=True)

    plsc.subcore_barrier()

    r0 = sid * rows_per_tile
    pltpu.sync_copy(
        acc_sh.at[pl.ds(r0, rows_per_tile)],
        acc_out.at[pl.ds(cid * n_nodes + r0, rows_per_tile)])
    if with_deg:
      @pl.when(sid == 0)
      def _():
        pltpu.sync_copy(deg_sh, deg_out.at[pl.ds(cid * n_nodes, n_nodes)])

  return pl.kernel(body, out_type=out_type, mesh=mesh, scratch_types=scratch)


def _prep_kernel(x_ref, w1a_ref, w1b_ref, wr1_ref, b1_ref, ab_ref, root_ref):
  x = x_ref[...]
  ab_ref[:, :16] = jnp.dot(x, w1a_ref[...], preferred_element_type=jnp.float32)
  ab_ref[:, 16:] = jnp.dot(x, w1b_ref[...], preferred_element_type=jnp.float32)
  root_ref[...] = (
      jnp.dot(x, wr1_ref[...], preferred_element_type=jnp.float32)
      + b1_ref[...])


def _mid_kernel(acc_ref, deg_ref, root_ref, w2a_ref, w2b_ref, wr2_ref, b2_ref,
                ab2_ref, root2_ref):
  agg = acc_ref[0] + acc_ref[1]
  d = jnp.maximum(deg_ref[0] + deg_ref[1], 1.0)
  h = agg / d + root_ref[...]
  h = jnp.where(h > 0.0, h, jnp.exp(h) - 1.0)
  ab2_ref[:, :16] = jnp.dot(h, w2a_ref[...], preferred_element_type=jnp.float32)
  ab2_ref[:, 16:] = jnp.dot(h, w2b_ref[...], preferred_element_type=jnp.float32)
  root2_ref[...] = (
      jnp.dot(h, wr2_ref[...], preferred_element_type=jnp.float32)
      + b2_ref[...])


def _final_kernel(acc_ref, deg_ref, root2_ref, out_ref):
  agg = acc_ref[0] + acc_ref[1]
  d = jnp.maximum(deg_ref[0] + deg_ref[1], 1.0)
  o = agg / d + root2_ref[...]
  lane = lax.broadcasted_iota(jnp.int32, o.shape, 1)
  o = jnp.where(lane >= 7, -1e30, o)
  m = jnp.max(o, axis=1, keepdims=True)
  s = jnp.log(jnp.sum(jnp.exp(o - m), axis=1, keepdims=True))
  out_ref[...] = (o - m) - s


def kernel(x, edge_index, edge_attr, W1, Wroot1, b1, W2, Wroot2, b2):
  n, f_in = x.shape
  e = edge_index.shape[1]
  hid = Wroot1.shape[1]
  c = Wroot2.shape[1]

  src = edge_index[0].astype(jnp.int32)
  dst = edge_index[1].astype(jnp.int32)
  p = edge_attr[:, 0]

  w2a = jnp.zeros((hid, 16), jnp.float32).at[:, :c].set(W2[0])
  w2b = jnp.zeros((hid, 16), jnp.float32).at[:, :c].set(W2[1])
  wr2 = jnp.zeros((hid, 16), jnp.float32).at[:, :c].set(Wroot2)
  b2p = jnp.zeros((16,), jnp.float32).at[:c].set(b2)

  blk = 2000
  grid = n // blk

  ab1, root1 = pl.pallas_call(
      _prep_kernel,
      grid=(grid,),
      in_specs=[
          pl.BlockSpec((blk, f_in), lambda i: (i, 0)),
          pl.BlockSpec((f_in, hid), lambda i: (0, 0)),
          pl.BlockSpec((f_in, hid), lambda i: (0, 0)),
          pl.BlockSpec((f_in, hid), lambda i: (0, 0)),
          pl.BlockSpec((1, hid), lambda i: (0, 0)),
      ],
      out_specs=[
          pl.BlockSpec((blk, 2 * hid), lambda i: (i, 0)),
          pl.BlockSpec((blk, hid), lambda i: (i, 0)),
      ],
      out_shape=[
          jax.ShapeDtypeStruct((n, 2 * hid), jnp.float32),
          jax.ShapeDtypeStruct((n, hid), jnp.float32),
      ],
  )(x, W1[0], W1[1], Wroot1, b1.reshape(1, hid))

  edge1 = _make_edge_pass(n, e, with_deg=True)
  acc1, deg = edge1(ab1, src, dst, p)
  acc1 = acc1.reshape(NCORE, n, hid)
  deg3 = deg.reshape(NCORE, n, 1)

  ab2, root2 = pl.pallas_call(
      _mid_kernel,
      grid=(grid,),
      in_specs=[
          pl.BlockSpec((NCORE, blk, hid), lambda i: (0, i, 0)),
          pl.BlockSpec((NCORE, blk, 1), lambda i: (0, i, 0)),
          pl.BlockSpec((blk, hid), lambda i: (i, 0)),
          pl.BlockSpec((hid, 16), lambda i: (0, 0)),
          pl.BlockSpec((hid, 16), lambda i: (0, 0)),
          pl.BlockSpec((hid, 16), lambda i: (0, 0)),
          pl.BlockSpec((1, 16), lambda i: (0, 0)),
      ],
      out_specs=[
          pl.BlockSpec((blk, 32), lambda i: (i, 0)),
          pl.BlockSpec((blk, 16), lambda i: (i, 0)),
      ],
      out_shape=[
          jax.ShapeDtypeStruct((n, 32), jnp.float32),
          jax.ShapeDtypeStruct((n, 16), jnp.float32),
      ],
  )(acc1, deg3, root1, w2a, w2b, wr2, b2p.reshape(1, 16))

  edge2 = _make_edge_pass(n, e, with_deg=False)
  (acc2,) = edge2(ab2, src, dst, p)
  acc2 = acc2.reshape(NCORE, n, 16)

  out = pl.pallas_call(
      _final_kernel,
      grid=(grid,),
      in_specs=[
          pl.BlockSpec((NCORE, blk, 16), lambda i: (0, i, 0)),
          pl.BlockSpec((NCORE, blk, 1), lambda i: (0, i, 0)),
          pl.BlockSpec((blk, 16), lambda i: (i, 0)),
      ],
      out_specs=pl.BlockSpec((blk, 16), lambda i: (i, 0)),
      out_shape=jax.ShapeDtypeStruct((n, 16), jnp.float32),
  )(acc2, deg3, root2)

  return out[:, :c]


# speedup vs baseline: 6.7528x; 6.7528x over previous
import functools

import jax
import jax.numpy as jnp
from jax import lax
from jax.experimental import pallas as pl
from jax.experimental.pallas import tpu as pltpu
from jax.experimental.pallas import tpu_sc as plsc

NW = 32
NCORE = 2
NSUB = 16
CHUNK = 80


def _scatter_add_rows(rows_ref, acc_ref, idx_ref):
  pltpu.sync_copy(rows_ref, acc_ref.at[idx_ref], add=True)


def _make_edge_pass(n_nodes, n_edges, with_deg):
  per_tile = n_edges // NW
  n_chunks = per_tile // CHUNK
  ztiles = 5
  rows_per_tile = n_nodes // ztiles
  mesh = plsc.VectorSubcoreMesh(
      core_axis_name="c", subcore_axis_name="s",
      num_cores=NCORE, num_subcores=NSUB)

  out_type = [jax.ShapeDtypeStruct((NCORE * n_nodes, 16), jnp.float32)]
  scratch = [
      pltpu.VMEM((per_tile,), jnp.int32),
      pltpu.VMEM((per_tile,), jnp.int32),
      pltpu.VMEM((per_tile,), jnp.float32),
      pltpu.VMEM((CHUNK,), jnp.int32),
      pltpu.VMEM((CHUNK,), jnp.int32),
      pltpu.VMEM((CHUNK, 32), jnp.float32),
      pltpu.VMEM((CHUNK, 16), jnp.float32),
      pltpu.VMEM((rows_per_tile, 16), jnp.float32),
      pltpu.VMEM_SHARED((n_nodes, 16), jnp.float32),
      pltpu.SemaphoreType.DMA,
  ]
  if with_deg:
    out_type.append(jax.ShapeDtypeStruct((NCORE * n_nodes, 16), jnp.float32))
    scratch += [
        pltpu.VMEM((CHUNK, 16), jnp.float32),
        pltpu.VMEM_SHARED((n_nodes, 16), jnp.float32),
    ]

  def body(ab_hbm, src_hbm, dst_hbm, p_hbm, acc_out, *rest):
    if with_deg:
      (deg_out, src_t, dst_t, p_t, idxr, idxw, rows_v, msg_v, zb, acc_sh, sem,
       ones_v, deg_sh) = rest
    else:
      (src_t, dst_t, p_t, idxr, idxw, rows_v, msg_v, zb, acc_sh, sem) = rest
      deg_out = ones_v = deg_sh = None

    cid = lax.axis_index("c")
    sid = lax.axis_index("s")
    wid = cid * NSUB + sid
    ebase = wid * per_tile

    pltpu.sync_copy(src_hbm.at[pl.ds(ebase, per_tile)], src_t)
    pltpu.sync_copy(dst_hbm.at[pl.ds(ebase, per_tile)], dst_t)
    pltpu.sync_copy(p_hbm.at[pl.ds(ebase, per_tile)], p_t)

    @pl.when(sid < ztiles)
    def _():
      @functools.partial(lax.fori_loop, 0, rows_per_tile, init_val=None)
      def _(i, _):
        zb[i, :] = jnp.zeros((16,), jnp.float32)

      pltpu.sync_copy(zb, acc_sh.at[pl.ds(sid * rows_per_tile, rows_per_tile)])
      if with_deg:
        pltpu.sync_copy(
            zb, deg_sh.at[pl.ds(sid * rows_per_tile, rows_per_tile)])

    if with_deg:
      @functools.partial(lax.fori_loop, 0, CHUNK, init_val=None)
      def _(i, _):
        ones_v[i, :] = jnp.ones((16,), jnp.float32)

    plsc.subcore_barrier()

    @functools.partial(lax.fori_loop, 0, n_chunks, init_val=None)
    def _(j, _):
      base = j * CHUNK
      for u in range(CHUNK // 16):
        idxr[pl.ds(u * 16, 16)] = src_t[pl.ds(base + u * 16, 16)]
        idxw[pl.ds(u * 16, 16)] = dst_t[pl.ds(base + u * 16, 16)]
      pltpu.async_copy(ab_hbm.at[idxr], rows_v, sem).wait()

      @functools.partial(lax.fori_loop, 0, CHUNK // 16, init_val=None)
      def _(g, _):
        pv = p_t[pl.ds(base + g * 16, 16)]
        pv = jnp.minimum(jnp.maximum(pv, 0.0), 1.0)
        for l in range(16):
          i = g * 16 + l
          a = rows_v[i, pl.ds(0, 16)]
          b = rows_v[i, pl.ds(16, 16)]
          msg_v[i, :] = a + pv[l] * (b - a)

      _scatter_add_rows(msg_v, acc_sh, idxw)
      if with_deg:
        _scatter_add_rows(ones_v, deg_sh, idxw)

    plsc.subcore_barrier()

    @pl.when(sid < ztiles)
    def _():
      r0 = sid * rows_per_tile
      pltpu.sync_copy(
          acc_sh.at[pl.ds(r0, rows_per_tile)],
          acc_out.at[pl.ds(cid * n_nodes + r0, rows_per_tile)])
    if with_deg:
      @pl.when(sid == 0)
      def _():
        pltpu.sync_copy(deg_sh, deg_out.at[pl.ds(cid * n_nodes, n_nodes)])

  return pl.kernel(
      body, out_type=out_type, mesh=mesh, scratch_types=scratch,
      compiler_params=pltpu.CompilerParams(use_tc_tiling_on_sc=False))


def _prep_kernel(x_ref, w1a_ref, w1b_ref, wr1_ref, b1_ref, ab_ref, root_ref):
  x = x_ref[...]
  ab_ref[:, :16] = jnp.dot(x, w1a_ref[...], preferred_element_type=jnp.float32)
  ab_ref[:, 16:] = jnp.dot(x, w1b_ref[...], preferred_element_type=jnp.float32)
  root_ref[...] = (
      jnp.dot(x, wr1_ref[...], preferred_element_type=jnp.float32)
      + b1_ref[...])


def _mid_kernel(acc_ref, deg_ref, root_ref, w2a_ref, w2b_ref, wr2_ref, b2_ref,
                ab2_ref, root2_ref):
  agg = acc_ref[0] + acc_ref[1]
  d = jnp.maximum(deg_ref[0, :, 0:1] + deg_ref[1, :, 0:1], 1.0)
  h = agg / d + root_ref[...]
  h = jnp.where(h > 0.0, h, jnp.exp(h) - 1.0)
  ab2_ref[:, :16] = jnp.dot(h, w2a_ref[...], preferred_element_type=jnp.float32)
  ab2_ref[:, 16:] = jnp.dot(h, w2b_ref[...], preferred_element_type=jnp.float32)
  root2_ref[...] = (
      jnp.dot(h, wr2_ref[...], preferred_element_type=jnp.float32)
      + b2_ref[...])


def _final_kernel(acc_ref, deg_ref, root2_ref, out_ref):
  agg = acc_ref[0] + acc_ref[1]
  d = jnp.maximum(deg_ref[0, :, 0:1] + deg_ref[1, :, 0:1], 1.0)
  o = agg / d + root2_ref[...]
  lane = lax.broadcasted_iota(jnp.int32, o.shape, 1)
  o = jnp.where(lane >= 7, -1e30, o)
  m = jnp.max(o, axis=1, keepdims=True)
  s = jnp.log(jnp.sum(jnp.exp(o - m), axis=1, keepdims=True))
  out_ref[...] = (o - m) - s


def kernel(x, edge_index, edge_attr, W1, Wroot1, b1, W2, Wroot2, b2):
  n, f_in = x.shape
  e = edge_index.shape[1]
  hid = Wroot1.shape[1]
  c = Wroot2.shape[1]

  src = edge_index[0].astype(jnp.int32)
  dst = edge_index[1].astype(jnp.int32)
  p = edge_attr[:, 0]

  w2a = jnp.zeros((hid, 16), jnp.float32).at[:, :c].set(W2[0])
  w2b = jnp.zeros((hid, 16), jnp.float32).at[:, :c].set(W2[1])
  wr2 = jnp.zeros((hid, 16), jnp.float32).at[:, :c].set(Wroot2)
  b2p = jnp.zeros((16,), jnp.float32).at[:c].set(b2)

  blk = 2000
  grid = n // blk

  ab1, root1 = pl.pallas_call(
      _prep_kernel,
      grid=(grid,),
      in_specs=[
          pl.BlockSpec((blk, f_in), lambda i: (i, 0)),
          pl.BlockSpec((f_in, hid), lambda i: (0, 0)),
          pl.BlockSpec((f_in, hid), lambda i: (0, 0)),
          pl.BlockSpec((f_in, hid), lambda i: (0, 0)),
          pl.BlockSpec((1, hid), lambda i: (0, 0)),
      ],
      out_specs=[
          pl.BlockSpec((blk, 2 * hid), lambda i: (i, 0)),
          pl.BlockSpec((blk, hid), lambda i: (i, 0)),
      ],
      out_shape=[
          jax.ShapeDtypeStruct((n, 2 * hid), jnp.float32),
          jax.ShapeDtypeStruct((n, hid), jnp.float32),
      ],
  )(x, W1[0], W1[1], Wroot1, b1.reshape(1, hid))

  edge1 = _make_edge_pass(n, e, with_deg=True)
  acc1, deg = edge1(ab1, src, dst, p)
  acc1 = acc1.reshape(NCORE, n, hid)
  deg3 = deg.reshape(NCORE, n, 16)

  ab2, root2 = pl.pallas_call(
      _mid_kernel,
      grid=(grid,),
      in_specs=[
          pl.BlockSpec((NCORE, blk, hid), lambda i: (0, i, 0)),
          pl.BlockSpec((NCORE, blk, 16), lambda i: (0, i, 0)),
          pl.BlockSpec((blk, hid), lambda i: (i, 0)),
          pl.BlockSpec((hid, 16), lambda i: (0, 0)),
          pl.BlockSpec((hid, 16), lambda i: (0, 0)),
          pl.BlockSpec((hid, 16), lambda i: (0, 0)),
          pl.BlockSpec((1, 16), lambda i: (0, 0)),
      ],
      out_specs=[
          pl.BlockSpec((blk, 32), lambda i: (i, 0)),
          pl.BlockSpec((blk, 16), lambda i: (i, 0)),
      ],
      out_shape=[
          jax.ShapeDtypeStruct((n, 32), jnp.float32),
          jax.ShapeDtypeStruct((n, 16), jnp.float32),
      ],
  )(acc1, deg3, root1, w2a, w2b, wr2, b2p.reshape(1, 16))

  edge2 = _make_edge_pass(n, e, with_deg=False)
  (acc2,) = edge2(ab2, src, dst, p)
  acc2 = acc2.reshape(NCORE, n, 16)

  out = pl.pallas_call(
      _final_kernel,
      grid=(grid,),
      in_specs=[
          pl.BlockSpec((NCORE, blk, 16), lambda i: (0, i, 0)),
          pl.BlockSpec((NCORE, blk, 16), lambda i: (0, i, 0)),
          pl.BlockSpec((blk, 16), lambda i: (i, 0)),
      ],
      out_specs=pl.BlockSpec((blk, 16), lambda i: (i, 0)),
      out_shape=jax.ShapeDtypeStruct((n, 16), jnp.float32),
  )(acc2, deg3, root2)

  return out[:, :c]


# speedup vs baseline: 8.9912x; 1.3315x over previous
import functools

import jax
import jax.numpy as jnp
from jax import lax
from jax.experimental import pallas as pl
from jax.experimental.pallas import tpu as pltpu
from jax.experimental.pallas import tpu_sc as plsc

NW = 32
NCORE = 2
NSUB = 16
CHUNK = 400


def _scatter_add_rows(rows_ref, acc_ref, idx_ref):
  pltpu.sync_copy(rows_ref, acc_ref.at[idx_ref], add=True)


def _make_edge_pass(n_nodes, n_edges, with_deg):
  per_tile = n_edges // NW
  n_chunks = per_tile // CHUNK
  ztiles = 5
  rows_per_tile = n_nodes // ztiles
  mesh = plsc.VectorSubcoreMesh(
      core_axis_name="c", subcore_axis_name="s",
      num_cores=NCORE, num_subcores=NSUB)

  out_type = [jax.ShapeDtypeStruct((NCORE * n_nodes, 16), jnp.float32)]
  scratch = [
      pltpu.VMEM((per_tile,), jnp.int32),
      pltpu.VMEM((per_tile,), jnp.int32),
      pltpu.VMEM((per_tile,), jnp.float32),
      pltpu.VMEM((CHUNK,), jnp.int32),
      pltpu.VMEM((CHUNK,), jnp.int32),
      pltpu.VMEM((CHUNK, 32), jnp.float32),
      pltpu.VMEM((CHUNK, 16), jnp.float32),
      pltpu.VMEM((rows_per_tile, 16), jnp.float32),
      pltpu.VMEM_SHARED((n_nodes, 16), jnp.float32),
      pltpu.SemaphoreType.DMA,
  ]
  if with_deg:
    out_type.append(jax.ShapeDtypeStruct((NCORE * n_nodes, 16), jnp.float32))
    scratch += [
        pltpu.VMEM((CHUNK, 16), jnp.float32),
        pltpu.VMEM_SHARED((n_nodes, 16), jnp.float32),
    ]

  def body(ab_hbm, src_hbm, dst_hbm, p_hbm, acc_out, *rest):
    if with_deg:
      (deg_out, src_t, dst_t, p_t, idxr, idxw, rows_v, msg_v, zb, acc_sh, sem,
       ones_v, deg_sh) = rest
    else:
      (src_t, dst_t, p_t, idxr, idxw, rows_v, msg_v, zb, acc_sh, sem) = rest
      deg_out = ones_v = deg_sh = None

    cid = lax.axis_index("c")
    sid = lax.axis_index("s")
    wid = cid * NSUB + sid
    ebase = wid * per_tile

    pltpu.sync_copy(src_hbm.at[pl.ds(ebase, per_tile)], src_t)
    pltpu.sync_copy(dst_hbm.at[pl.ds(ebase, per_tile)], dst_t)
    pltpu.sync_copy(p_hbm.at[pl.ds(ebase, per_tile)], p_t)

    @pl.when(sid < ztiles)
    def _():
      @functools.partial(lax.fori_loop, 0, rows_per_tile, init_val=None)
      def _(i, _):
        zb[i, :] = jnp.zeros((16,), jnp.float32)

      pltpu.sync_copy(zb, acc_sh.at[pl.ds(sid * rows_per_tile, rows_per_tile)])
      if with_deg:
        pltpu.sync_copy(
            zb, deg_sh.at[pl.ds(sid * rows_per_tile, rows_per_tile)])

    if with_deg:
      @functools.partial(lax.fori_loop, 0, CHUNK, init_val=None)
      def _(i, _):
        ones_v[i, :] = jnp.ones((16,), jnp.float32)

    plsc.subcore_barrier()

    @functools.partial(lax.fori_loop, 0, n_chunks, init_val=None)
    def _(j, _):
      base = j * CHUNK
      for u in range(CHUNK // 16):
        idxr[pl.ds(u * 16, 16)] = src_t[pl.ds(base + u * 16, 16)]
        idxw[pl.ds(u * 16, 16)] = dst_t[pl.ds(base + u * 16, 16)]
      pltpu.async_copy(ab_hbm.at[idxr], rows_v, sem).wait()

      @functools.partial(lax.fori_loop, 0, CHUNK // 16, init_val=None)
      def _(g, _):
        pv = p_t[pl.ds(base + g * 16, 16)]
        pv = jnp.minimum(jnp.maximum(pv, 0.0), 1.0)
        for l in range(16):
          i = g * 16 + l
          a = rows_v[i, pl.ds(0, 16)]
          b = rows_v[i, pl.ds(16, 16)]
          msg_v[i, :] = a + pv[l] * (b - a)

      _scatter_add_rows(msg_v, acc_sh, idxw)
      if with_deg:
        _scatter_add_rows(ones_v, deg_sh, idxw)

    plsc.subcore_barrier()

    @pl.when(sid < ztiles)
    def _():
      r0 = sid * rows_per_tile
      pltpu.sync_copy(
          acc_sh.at[pl.ds(r0, rows_per_tile)],
          acc_out.at[pl.ds(cid * n_nodes + r0, rows_per_tile)])
    if with_deg:
      @pl.when(sid == 0)
      def _():
        pltpu.sync_copy(deg_sh, deg_out.at[pl.ds(cid * n_nodes, n_nodes)])

  return pl.kernel(
      body, out_type=out_type, mesh=mesh, scratch_types=scratch,
      compiler_params=pltpu.CompilerParams(use_tc_tiling_on_sc=False))


def _prep_kernel(x_ref, w1a_ref, w1b_ref, wr1_ref, b1_ref, ab_ref, root_ref):
  x = x_ref[...]
  ab_ref[:, :16] = jnp.dot(x, w1a_ref[...], preferred_element_type=jnp.float32)
  ab_ref[:, 16:] = jnp.dot(x, w1b_ref[...], preferred_element_type=jnp.float32)
  root_ref[...] = (
      jnp.dot(x, wr1_ref[...], preferred_element_type=jnp.float32)
      + b1_ref[...])


def _mid_kernel(acc_ref, deg_ref, root_ref, w2a_ref, w2b_ref, wr2_ref, b2_ref,
                ab2_ref, root2_ref):
  agg = acc_ref[0] + acc_ref[1]
  d = jnp.maximum(deg_ref[0, :, 0:1] + deg_ref[1, :, 0:1], 1.0)
  h = agg / d + root_ref[...]
  h = jnp.where(h > 0.0, h, jnp.exp(h) - 1.0)
  ab2_ref[:, :16] = jnp.dot(h, w2a_ref[...], preferred_element_type=jnp.float32)
  ab2_ref[:, 16:] = jnp.dot(h, w2b_ref[...], preferred_element_type=jnp.float32)
  root2_ref[...] = (
      jnp.dot(h, wr2_ref[...], preferred_element_type=jnp.float32)
      + b2_ref[...])


def _final_kernel(acc_ref, deg_ref, root2_ref, out_ref):
  agg = acc_ref[0] + acc_ref[1]
  d = jnp.maximum(deg_ref[0, :, 0:1] + deg_ref[1, :, 0:1], 1.0)
  o = agg / d + root2_ref[...]
  lane = lax.broadcasted_iota(jnp.int32, o.shape, 1)
  o = jnp.where(lane >= 7, -1e30, o)
  m = jnp.max(o, axis=1, keepdims=True)
  s = jnp.log(jnp.sum(jnp.exp(o - m), axis=1, keepdims=True))
  out_ref[...] = (o - m) - s


def kernel(x, edge_index, edge_attr, W1, Wroot1, b1, W2, Wroot2, b2):
  n, f_in = x.shape
  e = edge_index.shape[1]
  hid = Wroot1.shape[1]
  c = Wroot2.shape[1]

  src = edge_index[0].astype(jnp.int32)
  dst = edge_index[1].astype(jnp.int32)
  p = edge_attr[:, 0]

  w2a = jnp.zeros((hid, 16), jnp.float32).at[:, :c].set(W2[0])
  w2b = jnp.zeros((hid, 16), jnp.float32).at[:, :c].set(W2[1])
  wr2 = jnp.zeros((hid, 16), jnp.float32).at[:, :c].set(Wroot2)
  b2p = jnp.zeros((16,), jnp.float32).at[:c].set(b2)

  blk = 2000
  grid = n // blk

  ab1, root1 = pl.pallas_call(
      _prep_kernel,
      grid=(grid,),
      in_specs=[
          pl.BlockSpec((blk, f_in), lambda i: (i, 0)),
          pl.BlockSpec((f_in, hid), lambda i: (0, 0)),
          pl.BlockSpec((f_in, hid), lambda i: (0, 0)),
          pl.BlockSpec((f_in, hid), lambda i: (0, 0)),
          pl.BlockSpec((1, hid), lambda i: (0, 0)),
      ],
      out_specs=[
          pl.BlockSpec((blk, 2 * hid), lambda i: (i, 0)),
          pl.BlockSpec((blk, hid), lambda i: (i, 0)),
      ],
      out_shape=[
          jax.ShapeDtypeStruct((n, 2 * hid), jnp.float32),
          jax.ShapeDtypeStruct((n, hid), jnp.float32),
      ],
  )(x, W1[0], W1[1], Wroot1, b1.reshape(1, hid))

  edge1 = _make_edge_pass(n, e, with_deg=True)
  acc1, deg = edge1(ab1, src, dst, p)
  acc1 = acc1.reshape(NCORE, n, hid)
  deg3 = deg.reshape(NCORE, n, 16)

  ab2, root2 = pl.pallas_call(
      _mid_kernel,
      grid=(grid,),
      in_specs=[
          pl.BlockSpec((NCORE, blk, hid), lambda i: (0, i, 0)),
          pl.BlockSpec((NCORE, blk, 16), lambda i: (0, i, 0)),
          pl.BlockSpec((blk, hid), lambda i: (i, 0)),
          pl.BlockSpec((hid, 16), lambda i: (0, 0)),
          pl.BlockSpec((hid, 16), lambda i: (0, 0)),
          pl.BlockSpec((hid, 16), lambda i: (0, 0)),
          pl.BlockSpec((1, 16), lambda i: (0, 0)),
      ],
      out_specs=[
          pl.BlockSpec((blk, 32), lambda i: (i, 0)),
          pl.BlockSpec((blk, 16), lambda i: (i, 0)),
      ],
      out_shape=[
          jax.ShapeDtypeStruct((n, 32), jnp.float32),
          jax.ShapeDtypeStruct((n, 16), jnp.float32),
      ],
  )(acc1, deg3, root1, w2a, w2b, wr2, b2p.reshape(1, 16))

  edge2 = _make_edge_pass(n, e, with_deg=False)
  (acc2,) = edge2(ab2, src, dst, p)
  acc2 = acc2.reshape(NCORE, n, 16)

  out = pl.pallas_call(
      _final_kernel,
      grid=(grid,),
      in_specs=[
          pl.BlockSpec((NCORE, blk, 16), lambda i: (0, i, 0)),
          pl.BlockSpec((NCORE, blk, 16), lambda i: (0, i, 0)),
          pl.BlockSpec((blk, 16), lambda i: (i, 0)),
      ],
      out_specs=pl.BlockSpec((blk, 16), lambda i: (i, 0)),
      out_shape=jax.ShapeDtypeStruct((n, 16), jnp.float32),
  )(acc2, deg3, root2)

  return out[:, :c]
